# baseline (speedup 1.0000x reference)
import jax
import jax.numpy as jnp
from jax import lax
from jax.experimental import pallas as pl
from jax.experimental.pallas import tpu as pltpu
from jax.experimental.pallas import tpu_sc as plsc

_NUM_TOKENS = 32768
_TOP_K = 8
_LANES = 16
_NUM_WORKERS = 32
_VECS = _NUM_TOKENS * _TOP_K // _LANES
_VPW = _VECS // _NUM_WORKERS


def _sc_body(idx_hbm, sc_hbm, out_s_hbm, out_i_hbm, idx_v, sc_v, os_v, oi_v):
    wid = lax.axis_index("s") * 2 + lax.axis_index("c")
    base = wid * _VPW
    pltpu.sync_copy(idx_hbm.at[pl.ds(base, _VPW)], idx_v)
    pltpu.sync_copy(sc_hbm.at[pl.ds(base, _VPW)], sc_v)

    lane = lax.iota(jnp.int32, _LANES)
    pos = lane & 7
    bias = (lane >> 3) << 9

    def body(j, carry):
        comp = idx_v[j] * 8 + pos + bias
        sk, sv = plsc.sort_key_val(comp, sc_v[j])
        os_v[j] = sv
        oi_v[j] = (sk & 7) >> 3
        return carry

    lax.fori_loop(0, _VPW, body, 0)

    pltpu.sync_copy(os_v, out_s_hbm.at[pl.ds(base, _VPW)])
    pltpu.sync_copy(oi_v, out_i_hbm.at[pl.ds(base, _VPW)])


def kernel(top_scores, selected_experts_indices):
    idx2 = selected_experts_indices.reshape(_VECS, _LANES)
    sc2 = top_scores.reshape(_VECS, _LANES)
    run = pl.kernel(
        _sc_body,
        out_type=(
            jax.ShapeDtypeStruct((_VECS, _LANES), jnp.float32),
            jax.ShapeDtypeStruct((_VECS, _LANES), jnp.int32),
        ),
        mesh=plsc.VectorSubcoreMesh(core_axis_name="c", subcore_axis_name="s"),
        compiler_params=pltpu.CompilerParams(
            needs_layout_passes=False, use_tc_tiling_on_sc=False
        ),
        scratch_types=[
            pltpu.VMEM((_VPW, _LANES), jnp.int32),
            pltpu.VMEM((_VPW, _LANES), jnp.float32),
            pltpu.VMEM((_VPW, _LANES), jnp.float32),
            pltpu.VMEM((_VPW, _LANES), jnp.int32),
        ],
    )
    os2, oi2 = run(idx2, sc2)
    return (
        os2.reshape(_NUM_TOKENS, _TOP_K),
        oi2.reshape(_NUM_TOKENS, _TOP_K),
    )


# speedup vs baseline: 1.0020x; 1.0020x over previous
import jax
import jax.numpy as jnp
from jax import lax
from jax.experimental import pallas as pl
from jax.experimental.pallas import tpu as pltpu
from jax.experimental.pallas import tpu_sc as plsc

_NUM_TOKENS = 32768
_TOP_K = 8
_LANES = 16
_NUM_WORKERS = 32
_VECS = _NUM_TOKENS * _TOP_K // _LANES
_VPW = _VECS // _NUM_WORKERS


def _sc_body(idx_hbm, sc_hbm, out_s_hbm, out_i_hbm, idx_v, sc_v, os_v, oi_v):
    wid = lax.axis_index("s") * 2 + lax.axis_index("c")
    base = wid * _VPW
    pltpu.sync_copy(idx_hbm.at[pl.ds(base, _VPW)], idx_v)
    pltpu.sync_copy(sc_hbm.at[pl.ds(base, _VPW)], sc_v)

    lane = lax.iota(jnp.int32, _LANES)
    pos = lane & 7
    bias = (lane >> 3) << 9

    def body(j, carry):
        comp = idx_v[j] * 8 + pos + bias
        sk, sv = plsc.sort_key_val(comp, sc_v[j])
        os_v[j] = sv
        oi_v[j] = (sk & 7) >> 3
        return carry

    lax.fori_loop(0, _VPW, body, 0)

    pltpu.sync_copy(os_v, out_s_hbm.at[pl.ds(base, _VPW)])
    pltpu.sync_copy(oi_v, out_i_hbm.at[pl.ds(base, _VPW)])


def kernel(top_scores, selected_experts_indices):
    idx2 = selected_experts_indices.reshape(_VECS, _LANES)
    sc2 = top_scores.reshape(_VECS, _LANES)
    run = pl.kernel(
        _sc_body,
        out_type=(
            jax.ShapeDtypeStruct((_VECS, _LANES), jnp.float32),
            jax.ShapeDtypeStruct((_VECS, _LANES), jnp.int32),
        ),
        mesh=plsc.VectorSubcoreMesh(core_axis_name="c", subcore_axis_name="s"),
        compiler_params=pltpu.CompilerParams(
            needs_layout_passes=False,
            use_tc_tiling_on_sc=False,
            skip_device_barrier=True,
        ),
        scratch_types=[
            pltpu.VMEM((_VPW, _LANES), jnp.int32),
            pltpu.VMEM((_VPW, _LANES), jnp.float32),
            pltpu.VMEM((_VPW, _LANES), jnp.float32),
            pltpu.VMEM((_VPW, _LANES), jnp.int32),
        ],
    )
    os2, oi2 = run(idx2, sc2)
    return (
        os2.reshape(_NUM_TOKENS, _TOP_K),
        oi2.reshape(_NUM_TOKENS, _TOP_K),
    )
